# baseline (speedup 1.0000x reference)
import dataclasses
import functools
import math

import jax
import jax.numpy as jnp
from jax import lax
from jax.experimental import pallas as pl
from jax.experimental.pallas import tpu as pltpu
from jax.experimental.pallas import tpu_sc as plsc

H = 128
EPS = 1e-5
SQH = math.sqrt(H)
NC, NS = 2, 16
NW = NC * NS
CHUNK = 128
NGBUF = 3
NOBUF = 2


def _fused_embed_ln(table, ids_arr, pe, gamma, beta, b, s):
    n = b * s
    pos_per_w = s // NW
    nch = n // (NW * CHUNK)
    ch_per_seq = pos_per_w // CHUNK
    mesh = plsc.VectorSubcoreMesh(core_axis_name="c", subcore_axis_name="s")
    cp = pltpu.CompilerParams()
    if "needs_layout_passes" in pltpu.CompilerParams.__dataclass_fields__:
        cp = dataclasses.replace(cp, needs_layout_passes=False)

    @functools.partial(
        pl.kernel,
        compiler_params=cp,
        out_type=jax.ShapeDtypeStruct((n, H), jnp.float32),
        mesh=mesh,
        scratch_types=[
            pltpu.VMEM((nch, CHUNK), jnp.int32),
            pltpu.VMEM((pos_per_w, H), jnp.float32),
            pltpu.VMEM((H,), jnp.float32),
            pltpu.VMEM((H,), jnp.float32),
            *[pltpu.VMEM((CHUNK, H), jnp.float32) for _ in range(NGBUF + NOBUF)],
            *[pltpu.SemaphoreType.DMA for _ in range(NGBUF + NOBUF)],
        ],
    )
    def body(table_hbm, idx_hbm, pe_hbm, gamma_hbm, beta_hbm, out_hbm,
             idx_v, pe_v, gamma_v, beta_v, *bufs_and_sems):
        gbufs = bufs_and_sems[:NGBUF]
        obufs = bufs_and_sems[NGBUF:NGBUF + NOBUF]
        gsems = bufs_and_sems[NGBUF + NOBUF:2 * NGBUF + NOBUF]
        wsems = bufs_and_sems[2 * NGBUF + NOBUF:]
        wid = lax.axis_index("s") * NC + lax.axis_index("c")
        idx_copies = []
        for k in range(nch):
            off = (k // ch_per_seq) * s + wid * pos_per_w \
                + (k % ch_per_seq) * CHUNK
            idx_copies.append(pltpu.async_copy(
                idx_hbm.at[pl.ds(off, CHUNK)], idx_v.at[k], gsems[0]))
        pe_copy = pltpu.async_copy(
            pe_hbm.at[pl.ds(wid * pos_per_w, pos_per_w)], pe_v, wsems[0])
        gm_copy = pltpu.async_copy(gamma_hbm, gamma_v, wsems[1])
        bt_copy = pltpu.async_copy(beta_hbm, beta_v, wsems[1])
        for c in idx_copies:
            c.wait()

        nsub = H // 16
        iota16 = lax.iota(jnp.int32, 16)

        def _tree_add(vs):
            vs = list(vs)
            while len(vs) > 1:
                vs = [a + b for a, b in zip(vs[0::2], vs[1::2])]
            return vs[0]

        def compute_chunk(buf, obuf, pe_base):
            @plsc.parallel_loop(0, CHUNK, unroll=1)
            def _(r):
                pe_r = pe_base + r
                xs = [buf[r, pl.ds(16 * i, 16)] * SQH
                      + pe_v[pe_r, pl.ds(16 * i, 16)] for i in range(nsub)]
                a1 = _tree_add(xs)
                a2 = _tree_add([x * x for x in xs])
                z = jnp.where(iota16 < 8,
                              a1 + jnp.flip(a1), a2 + jnp.flip(a2))
                p = plsc.cumsum(z)
                t1 = p.at[jnp.full((16,), 7, jnp.int32)].get(
                    mode="promise_in_bounds")
                t2 = p.at[jnp.full((16,), 15, jnp.int32)].get(
                    mode="promise_in_bounds") - t1
                mean = t1 * (1.0 / H)
                var = t2 * (1.0 / H) - mean * mean
                v = var + EPS
                iv = plsc.bitcast(v, jnp.int32)
                y = plsc.bitcast(
                    jnp.full((16,), 0x5F3759DF, jnp.int32) - (iv >> 1),
                    jnp.float32)
                rstd = y * (1.5 - 0.5 * v * y * y)
                for i in range(nsub):
                    obuf[r, pl.ds(16 * i, 16)] = \
                        (xs[i] - mean) * rstd * gvecs[i] + bvecs[i]

        gathers = [None] * nch
        writes = [None] * nch
        for k in range(min(NGBUF, nch)):
            gathers[k] = pltpu.async_copy(
                table_hbm.at[idx_v.at[k]], gbufs[k], gsems[k])
        pe_copy.wait()
        gm_copy.wait()
        bt_copy.wait()
        gvecs = [gamma_v[pl.ds(16 * i, 16)] for i in range(nsub)]
        bvecs = [beta_v[pl.ds(16 * i, 16)] for i in range(nsub)]
        for k in range(nch):
            p, q = k % NGBUF, k % NOBUF
            gathers[k].wait()
            if k >= NOBUF:
                writes[k - NOBUF].wait()
            compute_chunk(gbufs[p], obufs[q], (k % ch_per_seq) * CHUNK)
            if k + NGBUF < nch:
                gathers[k + NGBUF] = pltpu.async_copy(
                    table_hbm.at[idx_v.at[k + NGBUF]], gbufs[p], gsems[p])
            out_base = (k // ch_per_seq) * s + wid * pos_per_w \
                + (k % ch_per_seq) * CHUNK
            writes[k] = pltpu.async_copy(
                obufs[q], out_hbm.at[pl.ds(out_base, CHUNK)], wsems[q])
        for k in range(max(0, nch - NOBUF), nch):
            writes[k].wait()

    return body(table, ids_arr, pe, gamma, beta)


def kernel(token_ids, table, gamma, beta, pe):
    b, s = token_ids.shape
    ids_flat = token_ids.astype(jnp.int32).reshape(-1)
    out = _fused_embed_ln(table, ids_flat, pe[:s], gamma, beta, b, s)
    return out.reshape(b, s, H)


# speedup vs baseline: 1.0038x; 1.0038x over previous
import dataclasses
import functools
import math

import jax
import jax.numpy as jnp
from jax import lax
from jax.experimental import pallas as pl
from jax.experimental.pallas import tpu as pltpu
from jax.experimental.pallas import tpu_sc as plsc

H = 128
EPS = 1e-5
SQH = math.sqrt(H)
NC, NS = 2, 16
NW = NC * NS
CHUNK = 128
NGBUF = 3
NOBUF = 2


def _fused_embed_ln(table, ids_arr, pe, gamma, beta, b, s):
    n = b * s
    pos_per_w = s // NW
    nch = n // (NW * CHUNK)
    ch_per_seq = pos_per_w // CHUNK
    mesh = plsc.VectorSubcoreMesh(core_axis_name="c", subcore_axis_name="s")
    cp = pltpu.CompilerParams()
    if "needs_layout_passes" in pltpu.CompilerParams.__dataclass_fields__:
        cp = dataclasses.replace(cp, needs_layout_passes=False)

    @functools.partial(
        pl.kernel,
        compiler_params=cp,
        out_type=jax.ShapeDtypeStruct((n, H), jnp.float32),
        mesh=mesh,
        scratch_types=[
            pltpu.VMEM((nch, CHUNK), jnp.int32),
            pltpu.VMEM((pos_per_w, H), jnp.float32),
            pltpu.VMEM((H,), jnp.float32),
            pltpu.VMEM((H,), jnp.float32),
            *[pltpu.VMEM((CHUNK, H), jnp.float32) for _ in range(NGBUF + NOBUF)],
            *[pltpu.SemaphoreType.DMA for _ in range(NGBUF + NOBUF)],
        ],
    )
    def body(table_hbm, idx_hbm, pe_hbm, gamma_hbm, beta_hbm, out_hbm,
             idx_v, pe_v, gamma_v, beta_v, *bufs_and_sems):
        gbufs = bufs_and_sems[:NGBUF]
        obufs = bufs_and_sems[NGBUF:NGBUF + NOBUF]
        gsems = bufs_and_sems[NGBUF + NOBUF:2 * NGBUF + NOBUF]
        wsems = bufs_and_sems[2 * NGBUF + NOBUF:]
        wid = lax.axis_index("s") * NC + lax.axis_index("c")
        idx_copies = []
        for k in range(nch):
            off = (k // ch_per_seq) * s + wid * pos_per_w \
                + (k % ch_per_seq) * CHUNK
            idx_copies.append(pltpu.async_copy(
                idx_hbm.at[pl.ds(off, CHUNK)], idx_v.at[k], gsems[0]))
        pe_copy = pltpu.async_copy(
            pe_hbm.at[pl.ds(wid * pos_per_w, pos_per_w)], pe_v, wsems[0])
        gm_copy = pltpu.async_copy(gamma_hbm, gamma_v, wsems[1])
        bt_copy = pltpu.async_copy(beta_hbm, beta_v, wsems[1])
        for c in idx_copies:
            c.wait()

        nsub = H // 16
        iota16 = lax.iota(jnp.int32, 16)

        def _tree_add(vs):
            vs = list(vs)
            while len(vs) > 1:
                vs = [a + b for a, b in zip(vs[0::2], vs[1::2])]
            return vs[0]

        def compute_chunk(buf, obuf, pe_base):
            @plsc.parallel_loop(0, CHUNK, unroll=2)
            def _(r):
                pe_r = pe_base + r
                xs = [buf[r, pl.ds(16 * i, 16)] * SQH
                      + pe_v[pe_r, pl.ds(16 * i, 16)] for i in range(nsub)]
                t1 = lax.broadcast(jnp.sum(_tree_add(xs)), (16,))
                t2 = lax.broadcast(
                    jnp.sum(_tree_add([x * x for x in xs])), (16,))
                mean = t1 * (1.0 / H)
                var = t2 * (1.0 / H) - mean * mean
                v = var + EPS
                iv = plsc.bitcast(v, jnp.int32)
                y = plsc.bitcast(
                    jnp.full((16,), 0x5F3759DF, jnp.int32) - (iv >> 1),
                    jnp.float32)
                rstd = y * (1.5 - 0.5 * v * y * y)
                for i in range(nsub):
                    obuf[r, pl.ds(16 * i, 16)] = \
                        (xs[i] - mean) * rstd * gvecs[i] + bvecs[i]

        gathers = [None] * nch
        writes = [None] * nch
        for k in range(min(NGBUF, nch)):
            gathers[k] = pltpu.async_copy(
                table_hbm.at[idx_v.at[k]], gbufs[k], gsems[k])
        pe_copy.wait()
        gm_copy.wait()
        bt_copy.wait()
        gvecs = [gamma_v[pl.ds(16 * i, 16)] for i in range(nsub)]
        bvecs = [beta_v[pl.ds(16 * i, 16)] for i in range(nsub)]
        for k in range(nch):
            p, q = k % NGBUF, k % NOBUF
            gathers[k].wait()
            if k >= NOBUF:
                writes[k - NOBUF].wait()
            compute_chunk(gbufs[p], obufs[q], (k % ch_per_seq) * CHUNK)
            if k + NGBUF < nch:
                gathers[k + NGBUF] = pltpu.async_copy(
                    table_hbm.at[idx_v.at[k + NGBUF]], gbufs[p], gsems[p])
            out_base = (k // ch_per_seq) * s + wid * pos_per_w \
                + (k % ch_per_seq) * CHUNK
            writes[k] = pltpu.async_copy(
                obufs[q], out_hbm.at[pl.ds(out_base, CHUNK)], wsems[q])
        for k in range(max(0, nch - NOBUF), nch):
            writes[k].wait()

    return body(table, ids_arr, pe, gamma, beta)


def kernel(token_ids, table, gamma, beta, pe):
    b, s = token_ids.shape
    ids_flat = token_ids.astype(jnp.int32).reshape(-1)
    out = _fused_embed_ln(table, ids_flat, pe[:s], gamma, beta, b, s)
    return out.reshape(b, s, H)


# speedup vs baseline: 1.0352x; 1.0314x over previous
import dataclasses
import functools
import math

import jax
import jax.numpy as jnp
from jax import lax
from jax.experimental import pallas as pl
from jax.experimental.pallas import tpu as pltpu
from jax.experimental.pallas import tpu_sc as plsc

H = 128
EPS = 1e-5
SQH = math.sqrt(H)
NC, NS = 2, 16
NW = NC * NS
CHUNK = 128
NGBUF = 3
NOBUF = 2


def _fused_embed_ln(table, ids_arr, pe, gamma, beta, b, s):
    n = b * s
    pos_per_w = s // NW
    nch = n // (NW * CHUNK)
    ch_per_seq = pos_per_w // CHUNK
    mesh = plsc.VectorSubcoreMesh(core_axis_name="c", subcore_axis_name="s")
    cp = pltpu.CompilerParams()
    if "needs_layout_passes" in pltpu.CompilerParams.__dataclass_fields__:
        cp = dataclasses.replace(cp, needs_layout_passes=False)

    @functools.partial(
        pl.kernel,
        compiler_params=cp,
        out_type=jax.ShapeDtypeStruct((n, H), jnp.float32),
        mesh=mesh,
        scratch_types=[
            pltpu.VMEM((nch, CHUNK), jnp.int32),
            pltpu.VMEM((pos_per_w, H), jnp.float32),
            pltpu.VMEM((H,), jnp.float32),
            pltpu.VMEM((H,), jnp.float32),
            *[pltpu.VMEM((CHUNK, H), jnp.float32) for _ in range(NGBUF + NOBUF)],
            *[pltpu.SemaphoreType.DMA for _ in range(NGBUF + NOBUF)],
        ],
    )
    def body(table_hbm, idx_hbm, pe_hbm, gamma_hbm, beta_hbm, out_hbm,
             idx_v, pe_v, gamma_v, beta_v, *bufs_and_sems):
        gbufs = bufs_and_sems[:NGBUF]
        obufs = bufs_and_sems[NGBUF:NGBUF + NOBUF]
        gsems = bufs_and_sems[NGBUF + NOBUF:2 * NGBUF + NOBUF]
        wsems = bufs_and_sems[2 * NGBUF + NOBUF:]
        wid = lax.axis_index("s") * NC + lax.axis_index("c")
        idx_copies = []
        for k in range(nch):
            off = (k // ch_per_seq) * s + wid * pos_per_w \
                + (k % ch_per_seq) * CHUNK
            idx_copies.append(pltpu.async_copy(
                idx_hbm.at[pl.ds(off, CHUNK)], idx_v.at[k], gsems[0]))
        pe_copy = pltpu.async_copy(
            pe_hbm.at[pl.ds(wid * pos_per_w, pos_per_w)], pe_v, wsems[0])
        gm_copy = pltpu.async_copy(gamma_hbm, gamma_v, wsems[1])
        bt_copy = pltpu.async_copy(beta_hbm, beta_v, wsems[1])
        for c in idx_copies:
            c.wait()

        nsub = H // 16
        iota16 = lax.iota(jnp.int32, 16)

        def _tree_add(vs):
            vs = list(vs)
            while len(vs) > 1:
                vs = [a + b for a, b in zip(vs[0::2], vs[1::2])]
            return vs[0]

        def compute_chunk(buf, obuf, pe_base):
            @plsc.parallel_loop(0, CHUNK, unroll=1)
            def _(r):
                pe_r = pe_base + r
                xs = [buf[r, pl.ds(16 * i, 16)] * SQH
                      + pe_v[pe_r, pl.ds(16 * i, 16)] for i in range(nsub)]
                t1 = lax.broadcast(jnp.sum(_tree_add(xs)), (16,))
                t2 = lax.broadcast(
                    jnp.sum(_tree_add([x * x for x in xs])), (16,))
                mean = t1 * (1.0 / H)
                var = t2 * (1.0 / H) - mean * mean
                v = var + EPS
                iv = plsc.bitcast(v, jnp.int32)
                y = plsc.bitcast(
                    jnp.full((16,), 0x5F3759DF, jnp.int32) - (iv >> 1),
                    jnp.float32)
                rstd = y * (1.5 - 0.5 * v * y * y)
                for i in range(nsub):
                    obuf[r, pl.ds(16 * i, 16)] = \
                        (xs[i] - mean) * rstd * gvecs[i] + bvecs[i]

        gathers = [None] * nch
        writes = [None] * nch
        for k in range(min(NGBUF, nch)):
            gathers[k] = pltpu.async_copy(
                table_hbm.at[idx_v.at[k]], gbufs[k], gsems[k])
        pe_copy.wait()
        gm_copy.wait()
        bt_copy.wait()
        gvecs = [gamma_v[pl.ds(16 * i, 16)] for i in range(nsub)]
        bvecs = [beta_v[pl.ds(16 * i, 16)] for i in range(nsub)]
        for k in range(nch):
            p, q = k % NGBUF, k % NOBUF
            gathers[k].wait()
            if k >= NOBUF:
                writes[k - NOBUF].wait()
            compute_chunk(gbufs[p], obufs[q], (k % ch_per_seq) * CHUNK)
            if k + NGBUF < nch:
                gathers[k + NGBUF] = pltpu.async_copy(
                    table_hbm.at[idx_v.at[k + NGBUF]], gbufs[p], gsems[p])
            out_base = (k // ch_per_seq) * s + wid * pos_per_w \
                + (k % ch_per_seq) * CHUNK
            writes[k] = pltpu.async_copy(
                obufs[q], out_hbm.at[pl.ds(out_base, CHUNK)], wsems[q])
        for k in range(max(0, nch - NOBUF), nch):
            writes[k].wait()

    return body(table, ids_arr, pe, gamma, beta)


def kernel(token_ids, table, gamma, beta, pe):
    b, s = token_ids.shape
    ids_flat = token_ids.astype(jnp.int32).reshape(-1)
    out = _fused_embed_ln(table, ids_flat, pe[:s], gamma, beta, b, s)
    return out.reshape(b, s, H)
